# Initial kernel scaffold; baseline (speedup 1.0000x reference)
import jax
import jax.numpy as jnp
from jax.experimental import pallas as pl


def kernel(distilled_features, private_features, gauss_mean_W, gauss_mean_b, gauss_logvar_W, gauss_logvar_b, W1, att_src1, att_dst1, bias1, W2, att_src2, att_dst2, bias2):
    raise NotImplementedError("write your pallas kernel here")


# speedup vs baseline: 50.0537x; 50.0537x over previous
import functools

import jax
import jax.numpy as jnp
from jax.experimental import pallas as pl
from jax.experimental.pallas import tpu as pltpu

_INTERPRET = False

B = 128
SROWS = 512
NPRIV = 4
PAIRS = [(i, j) for i in range(4) for j in range(i + 1, 4)]
PAIR_IDX = {}
for _p, (_i, _j) in enumerate(PAIRS):
    PAIR_IDX[(_i, _j)] = _p
    PAIR_IDX[(_j, _i)] = _p
KL_T = 0.5
NEG = -1e30


def _leaky(x):
    return jnp.where(x >= 0, x, 0.2 * x)


def _fused_kernel(dist_ref, priv_ref, dg_ref, gw_ref, gb_ref, w1_ref, b1_ref,
                  w2_ref, b2_ref, a1cat_ref, a2cat_ref,
                  out_ref, kl_ref):
    g = pl.program_id(0)

    @pl.when(g < B)
    def _dense():
        ms, lvs = [], []
        for i in range(NPRIV):
            mv = jnp.dot(priv_ref[i], gw_ref[i],
                         preferred_element_type=jnp.float32) + gb_ref[i]
            ms.append(mv[:, :128])
            lvs.append(mv[:, 128:])
        es = [jnp.exp(lv) for lv in lvs]
        rs = [jnp.exp(-lv) for lv in lvs]
        kvals = []
        for (i, j) in PAIRS:
            d = ms[i] - ms[j]
            expr = (lvs[j] - lvs[i]) + (es[i] + d * d) * rs[j] - 1.0
            kvals.append(jnp.reshape(jnp.sum(expr) * (0.5 / SROWS), (1, 1)))
        kvals.append(jnp.zeros((1, 2), jnp.float32))
        kl_ref[pl.ds(g, 1), :] = jnp.concatenate(kvals, axis=1)

        x = dist_ref[...]
        xw1 = jnp.dot(x, w1_ref[...], preferred_element_type=jnp.float32)
        h1 = jnp.maximum(xw1 + b1_ref[...], 0.0)
        out_ref[...] = jnp.dot(h1, w2_ref[...],
                               preferred_element_type=jnp.float32) + b2_ref[...]

    @pl.when(g == B)
    def _graph():
        klT = kl_ref[...]
        mb = []
        for p in range(6):
            cf = (klT[:, p:p + 1] > KL_T).astype(jnp.float32)
            mb.append((1.0 - cf) * NEG)

        def mbias(i, j):
            return None if i == j else mb[PAIR_IDX[(i, j)]]

        F = [jnp.dot(dg_ref[i], w1_ref[...],
                     preferred_element_type=jnp.float32) for i in range(4)]
        P = [jnp.dot(F[i], a1cat_ref[...],
                     preferred_element_type=jnp.float32) for i in range(4)]

        def attend(src_cols, dst_cols, width, feats):
            H = src_cols[0].shape[1]
            mixed = []
            for j in range(4):
                alphas = []
                for i in range(4):
                    a = _leaky(src_cols[i] + dst_cols[j])
                    m = mbias(i, j)
                    if m is not None:
                        a = a + m
                    alphas.append(a)
                mx = jnp.maximum(jnp.maximum(alphas[0], alphas[1]),
                                 jnp.maximum(alphas[2], alphas[3]))
                exs = [jnp.exp(a - mx) for a in alphas]
                den = exs[0] + exs[1] + exs[2] + exs[3]
                acc = jnp.zeros_like(feats[0])
                for i in range(4):
                    c = exs[i] / den
                    if H == 1:
                        cexp = jnp.broadcast_to(c, (B, width))
                    else:
                        cexp = jnp.concatenate(
                            [jnp.broadcast_to(c[:, h:h + 1], (B, width))
                             for h in range(H)], axis=1)
                    acc = acc + cexp * feats[i]
                mixed.append(acc)
            return mixed

        src1 = [P[i][:, 0:4] for i in range(4)]
        dst1 = [P[j][:, 4:8] for j in range(4)]
        h1m = attend(src1, dst1, 64, F)
        xw2 = [jnp.dot(jnp.maximum(h1m[j] + b1_ref[...], 0.0), w2_ref[...],
                       preferred_element_type=jnp.float32) for j in range(4)]
        P2 = [jnp.dot(xw2[i], a2cat_ref[...],
                      preferred_element_type=jnp.float32) for i in range(4)]
        src2 = [P2[i][:, 0:1] for i in range(4)]
        dst2 = [P2[j][:, 1:2] for j in range(4)]
        outm = attend(src2, dst2, 128, xw2)

        rowi = jax.lax.broadcasted_iota(jnp.int32, (SROWS, B), 0)
        coli = jax.lax.broadcasted_iota(jnp.int32, (SROWS, B), 1)
        outg = jnp.zeros((SROWS, 128), jnp.float32)
        for j in range(4):
            Tj = (rowi == 4 * coli + j).astype(jnp.float32)
            outg = outg + jnp.dot(Tj, outm[j],
                                  preferred_element_type=jnp.float32)
        out_ref[...] = outg + b2_ref[...]


@jax.jit
def _run(dist2d, priv3d, dgraph, gwcat, gbcat, W1, b1, W2, b2, a1cat, a2cat):
    N = dist2d.shape[0]
    grid = (B + 1,)
    last = B - 1

    def tile_idx(g):
        return (jnp.where(g == B, 0, jnp.minimum(g, last)), 0)

    def ptile_idx(g):
        return (0, jnp.where(g == B, 0, jnp.minimum(g, last)), 0)

    out = pl.pallas_call(
        _fused_kernel,
        grid=grid,
        in_specs=[
            pl.BlockSpec((SROWS, 128), tile_idx),
            pl.BlockSpec((NPRIV, SROWS, 128), ptile_idx),
            pl.BlockSpec((NPRIV, B, 128), lambda g: (0, 0, 0)),
            pl.BlockSpec((NPRIV, 128, 256), lambda g: (0, 0, 0)),
            pl.BlockSpec((NPRIV, 1, 256), lambda g: (0, 0, 0)),
            pl.BlockSpec((128, 256), lambda g: (0, 0)),
            pl.BlockSpec((1, 256), lambda g: (0, 0)),
            pl.BlockSpec((256, 128), lambda g: (0, 0)),
            pl.BlockSpec((1, 128), lambda g: (0, 0)),
            pl.BlockSpec((256, 128), lambda g: (0, 0)),
            pl.BlockSpec((128, 128), lambda g: (0, 0)),
        ],
        out_specs=pl.BlockSpec((SROWS, 128), tile_idx),
        out_shape=jax.ShapeDtypeStruct((N, 128), jnp.float32),
        scratch_shapes=[pltpu.VMEM((B, 8), jnp.float32)],
        interpret=_INTERPRET,
    )(dist2d, priv3d, dgraph, gwcat, gbcat, W1, b1, W2, b2, a1cat, a2cat)
    return out


def kernel(distilled_features, private_features, gauss_mean_W, gauss_mean_b,
           gauss_logvar_W, gauss_logvar_b, W1, att_src1, att_dst1, bias1,
           W2, att_src2, att_dst2, bias2):
    bsz, s, d = distilled_features.shape
    dist2d = distilled_features.reshape(-1, d)
    priv3d = private_features.reshape(NPRIV, -1, 128)
    dgraph = dist2d[:SROWS].reshape(B, 4, d).transpose(1, 0, 2)
    gwcat = jnp.concatenate([gauss_mean_W, gauss_logvar_W], axis=2)
    gbcat = jnp.concatenate([gauss_mean_b, gauss_logvar_b],
                            axis=1).reshape(NPRIV, 1, 256)
    a1cat = jnp.zeros((256, 128), jnp.float32)
    for h in range(4):
        a1cat = a1cat.at[h * 64:(h + 1) * 64, h].set(att_src1[h])
        a1cat = a1cat.at[h * 64:(h + 1) * 64, 4 + h].set(att_dst1[h])
    a2cat = jnp.zeros((128, 128), jnp.float32)
    a2cat = a2cat.at[:, 0].set(att_src2[0]).at[:, 1].set(att_dst2[0])
    out = _run(dist2d, priv3d, dgraph, gwcat, gbcat,
               W1, bias1.reshape(1, -1), W2, bias2.reshape(1, -1),
               a1cat, a2cat)
    return out.reshape(bsz, s, -1)


# speedup vs baseline: 62.4432x; 1.2475x over previous
import functools

import jax
import jax.numpy as jnp
from jax.experimental import pallas as pl
from jax.experimental.pallas import tpu as pltpu

_INTERPRET = False

B = 128
SROWS = 512
NB = 2
TROWS = SROWS * NB
NSTEPS = B // NB
NPRIV = 4
PAIRS = [(i, j) for i in range(4) for j in range(i + 1, 4)]
PAIR_IDX = {}
for _p, (_i, _j) in enumerate(PAIRS):
    PAIR_IDX[(_i, _j)] = _p
    PAIR_IDX[(_j, _i)] = _p
KL_T = 0.5
NEG = -1e30


def _leaky(x):
    return jnp.where(x >= 0, x, 0.2 * x)


def _fused_kernel(dist_ref, priv_ref, dg_ref, gw_ref, gb_ref, w1_ref, b1_ref,
                  w2_ref, b2_ref, a1cat_ref, a2cat_ref,
                  out_ref, kl_ref):
    g = pl.program_id(0)

    @pl.when(g < NSTEPS)
    def _dense():
        ms, lvs = [], []
        for i in range(NPRIV):
            mv = jnp.dot(priv_ref[i], gw_ref[i],
                         preferred_element_type=jnp.float32) + gb_ref[i]
            ms.append(mv[:, :128])
            lvs.append(mv[:, 128:])
        es = [jnp.exp(lv) for lv in lvs]
        rs = [jnp.exp(-lv) for lv in lvs]
        rows = []
        for lb in range(NB):
            lo = lb * SROWS
            kvals = []
            for (i, j) in PAIRS:
                d = ms[i][lo:lo + SROWS] - ms[j][lo:lo + SROWS]
                expr = ((lvs[j][lo:lo + SROWS] - lvs[i][lo:lo + SROWS])
                        + (es[i][lo:lo + SROWS] + d * d) * rs[j][lo:lo + SROWS]
                        - 1.0)
                kvals.append(jnp.reshape(jnp.sum(expr) * (0.5 / SROWS), (1, 1)))
            kvals.append(jnp.zeros((1, 2), jnp.float32))
            rows.append(jnp.concatenate(kvals, axis=1))
        kl_ref[pl.ds(g * NB, NB), :] = jnp.concatenate(rows, axis=0)

        x = dist_ref[...]
        xw1 = jnp.dot(x, w1_ref[...], preferred_element_type=jnp.float32)
        h1 = jnp.maximum(xw1 + b1_ref[...], 0.0)
        out_ref[...] = jnp.dot(h1, w2_ref[...],
                               preferred_element_type=jnp.float32) + b2_ref[...]

    @pl.when(g == NSTEPS)
    def _graph():
        if NB > 1:
            x = dist_ref[SROWS:, :]
            xw1 = jnp.dot(x, w1_ref[...], preferred_element_type=jnp.float32)
            h1 = jnp.maximum(xw1 + b1_ref[...], 0.0)
            out_ref[SROWS:, :] = jnp.dot(
                h1, w2_ref[...], preferred_element_type=jnp.float32) + b2_ref[...]

        klT = kl_ref[...]
        mb = []
        for p in range(6):
            cf = (klT[:, p:p + 1] > KL_T).astype(jnp.float32)
            mb.append((1.0 - cf) * NEG)

        def mbias(i, j):
            return None if i == j else mb[PAIR_IDX[(i, j)]]

        F = [jnp.dot(dg_ref[i], w1_ref[...],
                     preferred_element_type=jnp.float32) for i in range(4)]
        P = [jnp.dot(F[i], a1cat_ref[...],
                     preferred_element_type=jnp.float32) for i in range(4)]

        def attend(src_cols, dst_cols, width, feats):
            H = src_cols[0].shape[1]
            mixed = []
            for j in range(4):
                alphas = []
                for i in range(4):
                    a = _leaky(src_cols[i] + dst_cols[j])
                    m = mbias(i, j)
                    if m is not None:
                        a = a + m
                    alphas.append(a)
                mx = jnp.maximum(jnp.maximum(alphas[0], alphas[1]),
                                 jnp.maximum(alphas[2], alphas[3]))
                exs = [jnp.exp(a - mx) for a in alphas]
                den = exs[0] + exs[1] + exs[2] + exs[3]
                acc = jnp.zeros_like(feats[0])
                for i in range(4):
                    c = exs[i] / den
                    if H == 1:
                        cexp = jnp.broadcast_to(c, (B, width))
                    else:
                        cexp = jnp.concatenate(
                            [jnp.broadcast_to(c[:, h:h + 1], (B, width))
                             for h in range(H)], axis=1)
                    acc = acc + cexp * feats[i]
                mixed.append(acc)
            return mixed

        src1 = [P[i][:, 0:4] for i in range(4)]
        dst1 = [P[j][:, 4:8] for j in range(4)]
        h1m = attend(src1, dst1, 64, F)
        xw2 = [jnp.dot(jnp.maximum(h1m[j] + b1_ref[...], 0.0), w2_ref[...],
                       preferred_element_type=jnp.float32) for j in range(4)]
        P2 = [jnp.dot(xw2[i], a2cat_ref[...],
                      preferred_element_type=jnp.float32) for i in range(4)]
        src2 = [P2[i][:, 0:1] for i in range(4)]
        dst2 = [P2[j][:, 1:2] for j in range(4)]
        outm = attend(src2, dst2, 128, xw2)

        rowi = jax.lax.broadcasted_iota(jnp.int32, (SROWS, B), 0)
        coli = jax.lax.broadcasted_iota(jnp.int32, (SROWS, B), 1)
        outg = jnp.zeros((SROWS, 128), jnp.float32)
        for j in range(4):
            Tj = (rowi == 4 * coli + j).astype(jnp.float32)
            outg = outg + jnp.dot(Tj, outm[j],
                                  preferred_element_type=jnp.float32)
        out_ref[:SROWS, :] = outg + b2_ref[...]


@jax.jit
def _run(dist2d, priv3d, dgraph, gwcat, gbcat, W1, b1, W2, b2, a1cat, a2cat):
    N = dist2d.shape[0]
    grid = (NSTEPS + 1,)
    last = NSTEPS - 1

    def tile_idx(g):
        return (jnp.where(g == NSTEPS, 0, jnp.minimum(g, last)), 0)

    def ptile_idx(g):
        return (0, jnp.where(g == NSTEPS, 0, jnp.minimum(g, last)), 0)

    out = pl.pallas_call(
        _fused_kernel,
        grid=grid,
        in_specs=[
            pl.BlockSpec((TROWS, 128), tile_idx),
            pl.BlockSpec((NPRIV, TROWS, 128), ptile_idx),
            pl.BlockSpec((NPRIV, B, 128), lambda g: (0, 0, 0)),
            pl.BlockSpec((NPRIV, 128, 256), lambda g: (0, 0, 0)),
            pl.BlockSpec((NPRIV, 1, 256), lambda g: (0, 0, 0)),
            pl.BlockSpec((128, 256), lambda g: (0, 0)),
            pl.BlockSpec((1, 256), lambda g: (0, 0)),
            pl.BlockSpec((256, 128), lambda g: (0, 0)),
            pl.BlockSpec((1, 128), lambda g: (0, 0)),
            pl.BlockSpec((256, 128), lambda g: (0, 0)),
            pl.BlockSpec((128, 128), lambda g: (0, 0)),
        ],
        out_specs=pl.BlockSpec((TROWS, 128), tile_idx),
        out_shape=jax.ShapeDtypeStruct((N, 128), jnp.float32),
        scratch_shapes=[pltpu.VMEM((B, 8), jnp.float32)],
        interpret=_INTERPRET,
    )(dist2d, priv3d, dgraph, gwcat, gbcat, W1, b1, W2, b2, a1cat, a2cat)
    return out


def kernel(distilled_features, private_features, gauss_mean_W, gauss_mean_b,
           gauss_logvar_W, gauss_logvar_b, W1, att_src1, att_dst1, bias1,
           W2, att_src2, att_dst2, bias2):
    bsz, s, d = distilled_features.shape
    dist2d = distilled_features.reshape(-1, d)
    priv3d = private_features.reshape(NPRIV, -1, 128)
    dgraph = dist2d[:SROWS].reshape(B, 4, d).transpose(1, 0, 2)
    gwcat = jnp.concatenate([gauss_mean_W, gauss_logvar_W], axis=2)
    gbcat = jnp.concatenate([gauss_mean_b, gauss_logvar_b],
                            axis=1).reshape(NPRIV, 1, 256)
    a1cat = jnp.zeros((256, 128), jnp.float32)
    for h in range(4):
        a1cat = a1cat.at[h * 64:(h + 1) * 64, h].set(att_src1[h])
        a1cat = a1cat.at[h * 64:(h + 1) * 64, 4 + h].set(att_dst1[h])
    a2cat = jnp.zeros((128, 128), jnp.float32)
    a2cat = a2cat.at[:, 0].set(att_src2[0]).at[:, 1].set(att_dst2[0])
    out = _run(dist2d, priv3d, dgraph, gwcat, gbcat,
               W1, bias1.reshape(1, -1), W2, bias2.reshape(1, -1),
               a1cat, a2cat)
    return out.reshape(bsz, s, -1)
